# baseline (speedup 1.0000x reference)
import functools

import jax
import jax.numpy as jnp
from jax import lax
from jax.experimental import pallas as pl
from jax.experimental.pallas import tpu as pltpu
from jax.experimental.pallas import tpu_sc as plsc


def _linear_stats_body(x_ref, wt_ref, b_ref, s_ref, ss_ref):
    y = jnp.dot(x_ref[...], wt_ref[...], preferred_element_type=jnp.float32)
    y = y + b_ref[...]

    @pl.when(pl.program_id(0) == 0)
    def _():
        s_ref[...] = jnp.zeros_like(s_ref)
        ss_ref[...] = jnp.zeros_like(ss_ref)

    s_ref[...] += jnp.sum(y, axis=0, keepdims=True)
    ss_ref[...] += jnp.sum(y * y, axis=0, keepdims=True)


def _linear_stats(x, wt, b, blk):
    R, Ci = x.shape
    Co = wt.shape[1]
    return pl.pallas_call(
        _linear_stats_body,
        grid=(R // blk,),
        in_specs=[
            pl.BlockSpec((blk, Ci), lambda i: (i, 0)),
            pl.BlockSpec((Ci, Co), lambda i: (0, 0)),
            pl.BlockSpec((1, Co), lambda i: (0, 0)),
        ],
        out_specs=[
            pl.BlockSpec((1, Co), lambda i: (0, 0)),
            pl.BlockSpec((1, Co), lambda i: (0, 0)),
        ],
        out_shape=[
            jax.ShapeDtypeStruct((1, Co), jnp.float32),
            jax.ShapeDtypeStruct((1, Co), jnp.float32),
        ],
    )(x, wt, b.reshape(1, Co))


def _mm_bn_relu_body(x_ref, wt_ref, b_ref, s_ref, ss_ref, g_ref, be_ref, o_ref, *, rows):
    y = jnp.dot(x_ref[...], wt_ref[...], preferred_element_type=jnp.float32)
    y = y + b_ref[...]
    m = s_ref[...] / rows
    v = ss_ref[...] / rows - m * m
    sc = g_ref[...] * lax.rsqrt(v + 1e-5)
    o_ref[...] = jnp.maximum(y * sc + (be_ref[...] - m * sc), 0.0)


def _mm_bn_relu(x, wt, b, s, ss, g, be, blk):
    R, Ci = x.shape
    Co = wt.shape[1]
    return pl.pallas_call(
        functools.partial(_mm_bn_relu_body, rows=float(R)),
        grid=(R // blk,),
        in_specs=[
            pl.BlockSpec((blk, Ci), lambda i: (i, 0)),
            pl.BlockSpec((Ci, Co), lambda i: (0, 0)),
            pl.BlockSpec((1, Co), lambda i: (0, 0)),
            pl.BlockSpec((1, Co), lambda i: (0, 0)),
            pl.BlockSpec((1, Co), lambda i: (0, 0)),
            pl.BlockSpec((1, Co), lambda i: (0, 0)),
            pl.BlockSpec((1, Co), lambda i: (0, 0)),
        ],
        out_specs=pl.BlockSpec((blk, Co), lambda i: (i, 0)),
        out_shape=jax.ShapeDtypeStruct((R, Co), jnp.float32),
    )(x, wt, b.reshape(1, Co), s, ss, g.reshape(1, Co), be.reshape(1, Co))


def _bn_knn_body(x_ref, wt_ref, b_ref, s_ref, ss_ref, g_ref, be_ref, p1_ref, p2t_ref,
                 h1_ref, idx_ref, w_ref, *, rows, n, m_blocks):
    i = pl.program_id(0)
    y = jnp.dot(x_ref[...], wt_ref[...], preferred_element_type=jnp.float32)
    y = y + b_ref[...]
    m = s_ref[...] / rows
    v = ss_ref[...] / rows - m * m
    sc = g_ref[...] * lax.rsqrt(v + 1e-5)
    h1_ref[...] = jnp.maximum(y * sc + (be_ref[...] - m * sc), 0.0)

    d = (p1_ref[:, 0:1] - p2t_ref[0, 0:1, :]) ** 2
    d = d + (p1_ref[:, 1:2] - p2t_ref[0, 1:2, :]) ** 2
    d = d + (p1_ref[:, 2:3] - p2t_ref[0, 2:3, :]) ** 2

    colf = lax.broadcasted_iota(jnp.int32, (1, n), 1).astype(jnp.float32)
    nf = jnp.float32(n)
    dists, sels = [], []
    for j in range(3):
        mval = jnp.min(d, axis=1, keepdims=True)
        sel = jnp.min(jnp.where(d == mval, colf, nf), axis=1, keepdims=True)
        dists.append(mval)
        sels.append(sel)
        if j < 2:
            d = jnp.where(colf == sel, jnp.float32(jnp.inf), d)

    inv = [1.0 / (t + 1e-8) for t in dists]
    tot = inv[0] + inv[1] + inv[2]
    w_ref[...] = jnp.concatenate([t / tot for t in inv], axis=1)
    bidx = i // m_blocks
    idx_ref[...] = jnp.concatenate(
        [s.astype(jnp.int32) for s in sels], axis=1) + bidx * n


def _bn_knn(x, wt, b, s, ss, g, be, p1f, p2t, blk):
    R, Ci = x.shape
    Co = wt.shape[1]
    Bb, _, N = p2t.shape
    m_blocks = R // Bb // blk
    return pl.pallas_call(
        functools.partial(_bn_knn_body, rows=float(R), n=N, m_blocks=m_blocks),
        grid=(R // blk,),
        in_specs=[
            pl.BlockSpec((blk, Ci), lambda i: (i, 0)),
            pl.BlockSpec((Ci, Co), lambda i: (0, 0)),
            pl.BlockSpec((1, Co), lambda i: (0, 0)),
            pl.BlockSpec((1, Co), lambda i: (0, 0)),
            pl.BlockSpec((1, Co), lambda i: (0, 0)),
            pl.BlockSpec((1, Co), lambda i: (0, 0)),
            pl.BlockSpec((1, Co), lambda i: (0, 0)),
            pl.BlockSpec((blk, 3), lambda i: (i, 0)),
            pl.BlockSpec((1, 3, N), lambda i, mb=m_blocks: (i // mb, 0, 0)),
        ],
        out_specs=[
            pl.BlockSpec((blk, Co), lambda i: (i, 0)),
            pl.BlockSpec((blk, 3), lambda i: (i, 0)),
            pl.BlockSpec((blk, 3), lambda i: (i, 0)),
        ],
        out_shape=[
            jax.ShapeDtypeStruct((R, Co), jnp.float32),
            jax.ShapeDtypeStruct((R, 3), jnp.int32),
            jax.ShapeDtypeStruct((R, 3), jnp.float32),
        ],
    )(x, wt, b.reshape(1, Co), s, ss, g.reshape(1, Co), be.reshape(1, Co), p1f, p2t)


_CH = 32


def _sc_combine_body(h2_hbm, h1_hbm, idx_hbm, w_hbm, out_hbm,
                     i0a, i1a, i2a, i0b, i1b, i2b,
                     r0a, r1a, r2a, r0b, r1b, r2b,
                     w0_v, w1_v, w2_v, h1_v, out_v, sema, semb,
                     *, qpw, co, bm):
    c = lax.axis_index("c")
    s = lax.axis_index("s")
    info = plsc.get_sparse_core_info()
    wid = s * info.num_cores + c
    base = wid * qpw
    nslice = co // 16
    nch = qpw // _CH
    ibufs = ((i0a, i1a, i2a), (i0b, i1b, i2b))
    rbufs = ((r0a, r1a, r2a), (r0b, r1b, r2b))
    sems = (sema, semb)

    def load_and_fire(ci, par):
        b0 = base + ci * _CH
        for j in range(3):
            pltpu.sync_copy(idx_hbm.at[pl.ds(j * bm + b0, _CH)], ibufs[par][j])
        for j in range(3):
            pltpu.async_copy(h2_hbm.at[ibufs[par][j]], rbufs[par][j], sems[par])

    load_and_fire(0, 0)

    def pair_body(k, carry):
        for par in (0, 1):
            ci = 2 * k + par
            nxt = ci + 1

            @pl.when(nxt < nch)
            def _():
                load_and_fire(nxt, par ^ 1)

            for j in range(3):
                pltpu.make_async_copy(
                    h2_hbm.at[ibufs[par][j]], rbufs[par][j], sems[par]).wait()

            b0 = base + ci * _CH
            pltpu.sync_copy(w_hbm.at[pl.ds(16 * b0, 16 * _CH)], w0_v)
            pltpu.sync_copy(w_hbm.at[pl.ds(16 * (bm + b0), 16 * _CH)], w1_v)
            pltpu.sync_copy(w_hbm.at[pl.ds(16 * (2 * bm + b0), 16 * _CH)], w2_v)
            pltpu.sync_copy(h1_hbm.at[pl.ds(b0, _CH)], h1_v)
            r0, r1, r2 = rbufs[par]

            def q_body(q, qc):
                wsl = pl.ds(q * 16, 16)
                w0 = w0_v[wsl]
                w1 = w1_v[wsl]
                w2 = w2_v[wsl]
                for t in range(nslice):
                    sl = pl.ds(t * 16, 16)
                    acc = h1_v[q, sl]
                    acc = acc + w0 * r0[q, sl]
                    acc = acc + w1 * r1[q, sl]
                    acc = acc + w2 * r2[q, sl]
                    out_v[q, sl] = acc
                return qc

            lax.fori_loop(0, _CH, q_body, 0)
            pltpu.sync_copy(out_v, out_hbm.at[pl.ds(b0, _CH)])
        return carry

    lax.fori_loop(0, nch // 2, pair_body, 0)


def _sc_combine(h2, h1, idxf, wf):
    BM, Co = h1.shape
    info = plsc.get_sparse_core_info()
    nw = info.num_cores * info.num_subcores
    qpw = BM // nw
    mesh = plsc.VectorSubcoreMesh(core_axis_name="c", subcore_axis_name="s")
    idx_t = pltpu.VMEM((_CH,), jnp.int32)
    row_t = pltpu.VMEM((_CH, Co), jnp.float32)
    wv_t = pltpu.VMEM((16 * _CH,), jnp.float32)
    return pl.kernel(
        functools.partial(_sc_combine_body, qpw=qpw, co=Co, bm=BM),
        mesh=mesh,
        out_type=jax.ShapeDtypeStruct((BM, Co), jnp.float32),
        scratch_types=[
            idx_t, idx_t, idx_t, idx_t, idx_t, idx_t,
            row_t, row_t, row_t, row_t, row_t, row_t,
            wv_t, wv_t, wv_t,
            row_t, row_t,
            pltpu.SemaphoreType.DMA,
            pltpu.SemaphoreType.DMA,
        ],
    )(h2, h1, idxf, wf)


def kernel(x1, p1, x2, p2, W1, b1, g1, be1, W2, b2, g2, be2):
    B, M, Co = x1.shape
    N, Ci = x2.shape[1], x2.shape[2]
    BM, BN = B * M, B * N

    x1f = x1.reshape(BM, Co)
    x2f = x2.reshape(BN, Ci)
    W1t, W2t = W1.T, W2.T

    s1, ss1 = _linear_stats(x1f, W1t, b1, blk=1024)
    s2, ss2 = _linear_stats(x2f, W2t, b2, blk=512)

    h2 = _mm_bn_relu(x2f, W2t, b2, s2, ss2, g2, be2, blk=512)

    p2t = jnp.transpose(p2, (0, 2, 1))
    h1, idx3, w3 = _bn_knn(x1f, W1t, b1, s1, ss1, g1, be1,
                           p1.reshape(BM, 3), p2t, blk=128)

    idxf = idx3.T.reshape(-1)
    wf = jnp.broadcast_to(w3.T.reshape(-1)[:, None], (3 * BM, 16)).reshape(-1)

    out = _sc_combine(h2, h1, idxf, wf)
    return out.reshape(B, M, Co)


# speedup vs baseline: 1.1608x; 1.1608x over previous
import functools

import jax
import jax.numpy as jnp
from jax import lax
from jax.experimental import pallas as pl
from jax.experimental.pallas import tpu as pltpu
from jax.experimental.pallas import tpu_sc as plsc


def _linear_stats_body(x_ref, wt_ref, b_ref, s_ref, ss_ref):
    y = jnp.dot(x_ref[...], wt_ref[...], preferred_element_type=jnp.float32)
    y = y + b_ref[...]

    @pl.when(pl.program_id(0) == 0)
    def _():
        s_ref[...] = jnp.zeros_like(s_ref)
        ss_ref[...] = jnp.zeros_like(ss_ref)

    s_ref[...] += jnp.sum(y, axis=0, keepdims=True)
    ss_ref[...] += jnp.sum(y * y, axis=0, keepdims=True)


def _linear_stats(x, wt, b, blk):
    R, Ci = x.shape
    Co = wt.shape[1]
    return pl.pallas_call(
        _linear_stats_body,
        grid=(R // blk,),
        in_specs=[
            pl.BlockSpec((blk, Ci), lambda i: (i, 0)),
            pl.BlockSpec((Ci, Co), lambda i: (0, 0)),
            pl.BlockSpec((1, Co), lambda i: (0, 0)),
        ],
        out_specs=[
            pl.BlockSpec((1, Co), lambda i: (0, 0)),
            pl.BlockSpec((1, Co), lambda i: (0, 0)),
        ],
        out_shape=[
            jax.ShapeDtypeStruct((1, Co), jnp.float32),
            jax.ShapeDtypeStruct((1, Co), jnp.float32),
        ],
    )(x, wt, b.reshape(1, Co))


def _mm_bn_relu_body(x_ref, wt_ref, b_ref, s_ref, ss_ref, g_ref, be_ref, o_ref, *, rows):
    y = jnp.dot(x_ref[...], wt_ref[...], preferred_element_type=jnp.float32)
    y = y + b_ref[...]
    m = s_ref[...] / rows
    v = ss_ref[...] / rows - m * m
    sc = g_ref[...] * lax.rsqrt(v + 1e-5)
    o_ref[...] = jnp.maximum(y * sc + (be_ref[...] - m * sc), 0.0)


def _mm_bn_relu(x, wt, b, s, ss, g, be, blk):
    R, Ci = x.shape
    Co = wt.shape[1]
    return pl.pallas_call(
        functools.partial(_mm_bn_relu_body, rows=float(R)),
        grid=(R // blk,),
        in_specs=[
            pl.BlockSpec((blk, Ci), lambda i: (i, 0)),
            pl.BlockSpec((Ci, Co), lambda i: (0, 0)),
            pl.BlockSpec((1, Co), lambda i: (0, 0)),
            pl.BlockSpec((1, Co), lambda i: (0, 0)),
            pl.BlockSpec((1, Co), lambda i: (0, 0)),
            pl.BlockSpec((1, Co), lambda i: (0, 0)),
            pl.BlockSpec((1, Co), lambda i: (0, 0)),
        ],
        out_specs=pl.BlockSpec((blk, Co), lambda i: (i, 0)),
        out_shape=jax.ShapeDtypeStruct((R, Co), jnp.float32),
    )(x, wt, b.reshape(1, Co), s, ss, g.reshape(1, Co), be.reshape(1, Co))


def _bn_knn_body(x_ref, wt_ref, b_ref, s_ref, ss_ref, g_ref, be_ref, p1_ref, p2t_ref,
                 h1_ref, idx_ref, w_ref, *, rows, n, m_blocks):
    i = pl.program_id(0)
    y = jnp.dot(x_ref[...], wt_ref[...], preferred_element_type=jnp.float32)
    y = y + b_ref[...]
    m = s_ref[...] / rows
    v = ss_ref[...] / rows - m * m
    sc = g_ref[...] * lax.rsqrt(v + 1e-5)
    h1_ref[...] = jnp.maximum(y * sc + (be_ref[...] - m * sc), 0.0)

    d = (p1_ref[:, 0:1] - p2t_ref[0, 0:1, :]) ** 2
    d = d + (p1_ref[:, 1:2] - p2t_ref[0, 1:2, :]) ** 2
    d = d + (p1_ref[:, 2:3] - p2t_ref[0, 2:3, :]) ** 2

    colf = lax.broadcasted_iota(jnp.int32, (1, n), 1).astype(jnp.float32)
    nf = jnp.float32(n)
    dists, sels = [], []
    for j in range(3):
        mval = jnp.min(d, axis=1, keepdims=True)
        sel = jnp.min(jnp.where(d == mval, colf, nf), axis=1, keepdims=True)
        dists.append(mval)
        sels.append(sel)
        if j < 2:
            d = jnp.where(colf == sel, jnp.float32(jnp.inf), d)

    inv = [1.0 / (t + 1e-8) for t in dists]
    tot = inv[0] + inv[1] + inv[2]
    w_ref[...] = jnp.concatenate(
        [jnp.broadcast_to(t / tot, (t.shape[0], 16)) for t in inv], axis=1)
    bidx = i // m_blocks
    idx_ref[...] = jnp.concatenate(
        [s.astype(jnp.int32) for s in sels], axis=1) + bidx * n


def _bn_knn(x, wt, b, s, ss, g, be, p1f, p2t, blk):
    R, Ci = x.shape
    Co = wt.shape[1]
    Bb, _, N = p2t.shape
    m_blocks = R // Bb // blk
    return pl.pallas_call(
        functools.partial(_bn_knn_body, rows=float(R), n=N, m_blocks=m_blocks),
        grid=(R // blk,),
        in_specs=[
            pl.BlockSpec((blk, Ci), lambda i: (i, 0)),
            pl.BlockSpec((Ci, Co), lambda i: (0, 0)),
            pl.BlockSpec((1, Co), lambda i: (0, 0)),
            pl.BlockSpec((1, Co), lambda i: (0, 0)),
            pl.BlockSpec((1, Co), lambda i: (0, 0)),
            pl.BlockSpec((1, Co), lambda i: (0, 0)),
            pl.BlockSpec((1, Co), lambda i: (0, 0)),
            pl.BlockSpec((blk, 3), lambda i: (i, 0)),
            pl.BlockSpec((1, 3, N), lambda i, mb=m_blocks: (i // mb, 0, 0)),
        ],
        out_specs=[
            pl.BlockSpec((blk, Co), lambda i: (i, 0)),
            pl.BlockSpec((blk, 3), lambda i: (i, 0)),
            pl.BlockSpec((blk, 48), lambda i: (i, 0)),
        ],
        out_shape=[
            jax.ShapeDtypeStruct((R, Co), jnp.float32),
            jax.ShapeDtypeStruct((R, 3), jnp.int32),
            jax.ShapeDtypeStruct((R, 48), jnp.float32),
        ],
    )(x, wt, b.reshape(1, Co), s, ss, g.reshape(1, Co), be.reshape(1, Co), p1f, p2t)


_CH = 32


def _sc_combine_body(h2_hbm, h1_hbm, idx_hbm, w_hbm, out_hbm,
                     idx_v, ra, rb, wa, wb, h1a, h1b, outa, outb,
                     gsa, gsb, lsa, lsb, osa, osb, *, qpw, co):
    c = lax.axis_index("c")
    s = lax.axis_index("s")
    info = plsc.get_sparse_core_info()
    wid = s * info.num_cores + c
    base = wid * qpw
    nslice = co // 16
    nch = qpw // _CH
    rbufs, wbufs, h1bufs, obufs = (ra, rb), (wa, wb), (h1a, h1b), (outa, outb)
    gsems, lsems, osems = (gsa, gsb), (lsa, lsb), (osa, osb)

    pltpu.sync_copy(idx_hbm.at[pl.ds(3 * base, 3 * qpw)], idx_v)

    def fire(ci, par):
        b0 = base + ci * _CH
        pltpu.async_copy(h2_hbm.at[idx_v.at[pl.ds(ci * (3 * _CH), 3 * _CH)]],
                         rbufs[par], gsems[par])
        pltpu.async_copy(w_hbm.at[pl.ds(48 * b0, 48 * _CH)], wbufs[par], lsems[par])
        pltpu.async_copy(h1_hbm.at[pl.ds(b0, _CH)], h1bufs[par], lsems[par])

    fire(0, 0)

    def pair_body(k, carry):
        for par in (0, 1):
            ci = 2 * k + par
            b0 = base + ci * _CH

            @pl.when(ci + 1 < nch)
            def _():
                fire(ci + 1, par ^ 1)

            pltpu.make_async_copy(
                h2_hbm.at[idx_v.at[pl.ds(ci * (3 * _CH), 3 * _CH)]],
                rbufs[par], gsems[par]).wait()
            pltpu.make_async_copy(
                w_hbm.at[pl.ds(48 * b0, 48 * _CH)], wbufs[par], lsems[par]).wait()
            pltpu.make_async_copy(
                h1_hbm.at[pl.ds(b0, _CH)], h1bufs[par], lsems[par]).wait()

            @pl.when(ci >= 2)
            def _():
                pltpu.make_async_copy(
                    obufs[par], out_hbm.at[pl.ds(b0, _CH)], osems[par]).wait()

            rv, wv, h1v, ov = rbufs[par], wbufs[par], h1bufs[par], obufs[par]

            def q_body(q, qc):
                w0 = wv[pl.ds(q * 48, 16)]
                w1 = wv[pl.ds(q * 48 + 16, 16)]
                w2 = wv[pl.ds(q * 48 + 32, 16)]
                q3 = q * 3
                for t in range(nslice):
                    sl = pl.ds(t * 16, 16)
                    acc = h1v[q, sl]
                    acc = acc + w0 * rv[q3, sl]
                    acc = acc + w1 * rv[q3 + 1, sl]
                    acc = acc + w2 * rv[q3 + 2, sl]
                    ov[q, sl] = acc
                return qc

            lax.fori_loop(0, _CH, q_body, 0)
            pltpu.async_copy(ov, out_hbm.at[pl.ds(b0, _CH)], osems[par])
        return carry

    lax.fori_loop(0, nch // 2, pair_body, 0)

    for par in (0, 1):
        b0 = base + (nch - 2 + par) * _CH
        pltpu.make_async_copy(
            obufs[par], out_hbm.at[pl.ds(b0, _CH)], osems[par]).wait()


def _sc_combine(h2, h1, idxf, wBf):
    BM, Co = h1.shape
    info = plsc.get_sparse_core_info()
    nw = info.num_cores * info.num_subcores
    qpw = BM // nw
    mesh = plsc.VectorSubcoreMesh(core_axis_name="c", subcore_axis_name="s")
    row_t = pltpu.VMEM((3 * _CH, Co), jnp.float32)
    wv_t = pltpu.VMEM((48 * _CH,), jnp.float32)
    ho_t = pltpu.VMEM((_CH, Co), jnp.float32)
    return pl.kernel(
        functools.partial(_sc_combine_body, qpw=qpw, co=Co),
        mesh=mesh,
        out_type=jax.ShapeDtypeStruct((BM, Co), jnp.float32),
        scratch_types=[
            pltpu.VMEM((3 * qpw,), jnp.int32),
            row_t, row_t, wv_t, wv_t, ho_t, ho_t, ho_t, ho_t,
            pltpu.SemaphoreType.DMA, pltpu.SemaphoreType.DMA,
            pltpu.SemaphoreType.DMA, pltpu.SemaphoreType.DMA,
            pltpu.SemaphoreType.DMA, pltpu.SemaphoreType.DMA,
        ],
    )(h2, h1, idxf, wBf)


def kernel(x1, p1, x2, p2, W1, b1, g1, be1, W2, b2, g2, be2):
    B, M, Co = x1.shape
    N, Ci = x2.shape[1], x2.shape[2]
    BM, BN = B * M, B * N

    x1f = x1.reshape(BM, Co)
    x2f = x2.reshape(BN, Ci)
    W1t, W2t = W1.T, W2.T

    s1, ss1 = _linear_stats(x1f, W1t, b1, blk=1024)
    s2, ss2 = _linear_stats(x2f, W2t, b2, blk=512)

    h2 = _mm_bn_relu(x2f, W2t, b2, s2, ss2, g2, be2, blk=512)

    p2t = jnp.transpose(p2, (0, 2, 1))
    h1, idx3, wB = _bn_knn(x1f, W1t, b1, s1, ss1, g1, be1,
                           p1.reshape(BM, 3), p2t, blk=128)

    out = _sc_combine(h2, h1, idx3.reshape(-1), wB.reshape(-1))
    return out.reshape(B, M, Co)
